# baseline (speedup 1.0000x reference)
import functools

import jax
import jax.numpy as jnp
from jax import lax
from jax.experimental import pallas as pl
from jax.experimental.pallas import tpu as pltpu
from jax.experimental.pallas import tpu_sc as plsc

VOCAB = 1000000
EMBED = 32
BATCH = 16384
HIST = 50

_NW = 32
_CH = 128
_NBT = BATCH // _CH
_NB = HIST * _NBT
_BPW = _NB // _NW
_TB = EMBED * _CH
_TSKEW = _CH + 8

_mesh = plsc.VectorSubcoreMesh(core_axis_name="c", subcore_axis_name="s")


@functools.partial(
    pl.kernel,
    mesh=_mesh,
    out_type=jax.ShapeDtypeStruct((HIST, EMBED // 8, _NBT, 8, _CH), jnp.float32),
    scratch_types=[
        pltpu.VMEM((_BPW, _CH), jnp.int32),
        [pltpu.VMEM((_CH, EMBED), jnp.float32) for _ in range(2)],
        [pltpu.VMEM((_TSKEW * EMBED,), jnp.float32) for _ in range(2)],
        [pltpu.SemaphoreType.DMA for _ in range(2)],
        [pltpu.SemaphoreType.DMA for _ in range(2)],
    ],
    compiler_params=pltpu.CompilerParams(
        use_tc_tiling_on_sc=False, needs_layout_passes=False
    ),
)
def _emb_lookup(idx_hbm, table_hbm, out_hbm, idx_v, rows, tbuf, gsem, wsem):
    wid = lax.axis_index("s") * 2 + lax.axis_index("c")
    base = wid * _BPW
    pltpu.sync_copy(idx_hbm.at[pl.ds(base, _BPW)], idx_v)

    lane_step = lax.iota(jnp.int32, 16) * _TSKEW

    def issue_gather(k, buf):
        pltpu.async_copy(table_hbm.at[idx_v.at[k]], rows[buf], gsem[buf])

    def wait_gather(buf):
        pltpu.make_async_copy(table_hbm.at[pl.ds(0, _CH)], rows[buf], gsem[buf]).wait()

    def transpose(buf):
        for l0 in range(0, _CH, 8):
            batch = []
            for i in range(8):
                l = l0 + i
                for half in range(2):
                    v = rows[buf][l, pl.ds(half * 16, 16)]
                    batch.append((lane_step + (half * 16 * _TSKEW + l), v))
            for addr, v in batch:
                plsc.store_scatter(tbuf[buf], [addr], v)

    def issue_write(k, buf):
        blk = base + k
        h = blk >> 7
        bt = blk & 127
        for e in range(EMBED):
            pltpu.async_copy(
                tbuf[buf].at[pl.ds(e * _TSKEW, _CH)],
                out_hbm.at[h, e // 8, bt, e % 8],
                wsem[buf],
            )

    def wait_write(buf):
        for e in range(EMBED):
            pltpu.make_async_copy(
                out_hbm.at[0, 0, 0, 0],
                tbuf[buf].at[pl.ds(e * _TSKEW, _CH)],
                wsem[buf],
            ).wait()

    issue_gather(0, 0)
    issue_gather(1, 1)
    for buf in range(2):
        wait_gather(buf)
        transpose(buf)
        issue_write(buf, buf)
        issue_gather(buf + 2, buf)

    def outer(g, carry):
        for p in range(2):
            k = g * 2 + p
            wait_gather(p)
            wait_write(p)
            transpose(p)
            issue_write(k, p)
            issue_gather(k + 2, p)
        return carry

    lax.fori_loop(1, _BPW // 2 - 1, outer, None)

    for p in range(2):
        k = _BPW - 2 + p
        wait_gather(p)
        wait_write(p)
        transpose(p)
        issue_write(k, p)
    for p in range(2):
        wait_write(p)


def kernel(indices, embeddings):
    idx = indices.astype(jnp.int32).T.reshape(_NB, _CH)
    out5d = _emb_lookup(idx, embeddings)
    return out5d.transpose(2, 4, 0, 1, 3).reshape(BATCH, HIST, EMBED)


# speedup vs baseline: 1.0233x; 1.0233x over previous
import functools

import jax
import jax.numpy as jnp
from jax import lax
from jax.experimental import pallas as pl
from jax.experimental.pallas import tpu as pltpu
from jax.experimental.pallas import tpu_sc as plsc

VOCAB = 1000000
EMBED = 32
BATCH = 16384
HIST = 50

_NW = 32
_CH = 128
_NBT = BATCH // _CH
_NB = HIST * _NBT
_BPW = _NB // _NW
_TB = EMBED * _CH
_TSKEW = _CH + 8

_mesh = plsc.VectorSubcoreMesh(core_axis_name="c", subcore_axis_name="s")


@functools.partial(
    pl.kernel,
    mesh=_mesh,
    out_type=jax.ShapeDtypeStruct((HIST, EMBED // 8, _NBT, 8, _CH), jnp.float32),
    scratch_types=[
        pltpu.VMEM((_BPW, _CH), jnp.int32),
        [pltpu.VMEM((_CH, EMBED), jnp.float32) for _ in range(2)],
        [pltpu.VMEM((_TSKEW * EMBED,), jnp.float32) for _ in range(2)],
        [pltpu.SemaphoreType.DMA for _ in range(2)],
        [pltpu.SemaphoreType.DMA for _ in range(2)],
    ],
    compiler_params=pltpu.CompilerParams(
        use_tc_tiling_on_sc=False, needs_layout_passes=False
    ),
)
def _emb_lookup(idx_hbm, table_hbm, out_hbm, idx_v, rows, tbuf, gsem, wsem):
    wid = lax.axis_index("s") * 2 + lax.axis_index("c")
    base = wid * _BPW
    pltpu.sync_copy(idx_hbm.at[pl.ds(base, _BPW)], idx_v)

    lane_step = lax.iota(jnp.int32, 16) * _TSKEW

    def issue_gather(k, buf):
        pltpu.async_copy(table_hbm.at[idx_v.at[k]], rows[buf], gsem[buf])

    def wait_gather(buf):
        pltpu.make_async_copy(table_hbm.at[pl.ds(0, _CH)], rows[buf], gsem[buf]).wait()

    def transpose(buf):
        for l0 in range(0, _CH, 4):
            batch = []
            for i in range(4):
                l = l0 + i
                for half in range(2):
                    v = rows[buf][l, pl.ds(half * 16, 16)]
                    batch.append((lane_step + (half * 16 * _TSKEW + l), v))
            for addr, v in batch:
                plsc.store_scatter(tbuf[buf], [addr], v)

    def issue_write(k, buf):
        blk = base + k
        h = blk >> 7
        bt = blk & 127
        for e in range(EMBED):
            pltpu.async_copy(
                tbuf[buf].at[pl.ds(e * _TSKEW, _CH)],
                out_hbm.at[h, e // 8, bt, e % 8],
                wsem[buf],
            )

    def wait_write(buf):
        for e in range(EMBED):
            pltpu.make_async_copy(
                out_hbm.at[0, 0, 0, 0],
                tbuf[buf].at[pl.ds(e * _TSKEW, _CH)],
                wsem[buf],
            ).wait()

    issue_gather(0, 0)
    issue_gather(1, 1)
    for buf in range(2):
        wait_gather(buf)
        transpose(buf)
        issue_write(buf, buf)
        issue_gather(buf + 2, buf)

    def outer(g, carry):
        for p in range(2):
            k = g * 2 + p
            wait_gather(p)
            wait_write(p)
            transpose(p)
            issue_write(k, p)
            issue_gather(k + 2, p)
        return carry

    lax.fori_loop(1, _BPW // 2 - 1, outer, None)

    for p in range(2):
        k = _BPW - 2 + p
        wait_gather(p)
        wait_write(p)
        transpose(p)
        issue_write(k, p)
    for p in range(2):
        wait_write(p)


def kernel(indices, embeddings):
    idx = indices.astype(jnp.int32).T.reshape(_NB, _CH)
    out5d = _emb_lookup(idx, embeddings)
    return out5d.transpose(2, 4, 0, 1, 3).reshape(BATCH, HIST, EMBED)
